# baseline (speedup 1.0000x reference)
import functools

import jax
import jax.numpy as jnp
from jax import lax
from jax.experimental import pallas as pl
from jax.experimental.pallas import tpu as pltpu
from jax.experimental.pallas import tpu_sc as plsc

B = 16384
D = 128


_NC = 2
_NS = 16
_NW = _NC * _NS
_CHUNK = 128
_NSEG = 2
_SEG = B // _NSEG
_CHUNKS_PER_W = _SEG // _NW // _CHUNK


def _sc_gather_body(seg, uids_hbm, mids_hbm, ut_hbm, mt_hbm,
                    uout_hbm, mout_hbm,
                    idx0, idx1, rows0, rows1, gsem, wsem):
    wid = lax.axis_index("s") * _NC + lax.axis_index("c")
    seg_chunk0 = seg * (_SEG // _CHUNK)
    base_chunk = wid * _CHUNKS_PER_W

    rounds = []
    for tbl, ids, out in ((ut_hbm, uids_hbm, uout_hbm),
                          (mt_hbm, mids_hbm, mout_hbm)):
        for g in range(_CHUNKS_PER_W):
            crow = base_chunk + g
            rounds.append((ids, seg_chunk0 + crow, tbl, out, crow * _CHUNK))
    n = len(rounds)
    idx = (idx0, idx1)
    rows = (rows0, rows1)

    ids0, irow0, tbl0, _, _ = rounds[0]
    pltpu.sync_copy(ids0.at[irow0], idx[0])
    gathers = [pltpu.async_copy(tbl0.at[idx[0]], rows[0], gsem)]
    writes = []
    for r in range(n):
        cb = r % 2
        if r + 1 < n:
            nids, nirow, ntbl, _, _ = rounds[r + 1]
            nb = (r + 1) % 2
            pltpu.sync_copy(nids.at[nirow], idx[nb])
            if r >= 1:
                writes[r - 1].wait()
            gathers.append(pltpu.async_copy(ntbl.at[idx[nb]], rows[nb], gsem))
        gathers[r].wait()
        _, _, _, out, orow = rounds[r]
        writes.append(
            pltpu.async_copy(rows[cb], out.at[pl.ds(orow, _CHUNK)], wsem))
    writes[n - 2].wait()
    writes[n - 1].wait()


def _sc_gather(seg, uids2d, mids2d, user_table, movie_table):
    mesh = plsc.VectorSubcoreMesh(core_axis_name="c", subcore_axis_name="s")
    k = functools.partial(
        pl.kernel,
        mesh=mesh,
        out_type=[
            jax.ShapeDtypeStruct((_SEG, D), jnp.float32),
            jax.ShapeDtypeStruct((_SEG, D), jnp.float32),
        ],
        scratch_types=[
            pltpu.VMEM((_CHUNK,), jnp.int32),
            pltpu.VMEM((_CHUNK,), jnp.int32),
            pltpu.VMEM((_CHUNK, D), jnp.float32),
            pltpu.VMEM((_CHUNK, D), jnp.float32),
            pltpu.SemaphoreType.DMA,
            pltpu.SemaphoreType.DMA,
        ],
    )(functools.partial(_sc_gather_body, seg))
    return k(uids2d, mids2d, user_table, movie_table)


_BB = 2048
_OR = _BB // 128


def _dense_body(f_ref, u_ref, m_ref,
                gw_ref, gb_ref, ow_ref, ob_ref, at_ref, gt_ref,
                w1_ref, b1_ref, w2_ref, b2_ref, w3_ref, b3_ref, out_ref):
    f32 = jnp.float32
    dn = (((1,), (1,)), ((), ()))

    x = f_ref[...].astype(f32)
    g = jax.nn.relu(
        lax.dot_general(x[:, 0:18], gw_ref[...], dn, preferred_element_type=f32)
        + gb_ref[...])
    o = jax.nn.relu(
        lax.dot_general(x[:, 18:39], ow_ref[...], dn, preferred_element_type=f32)
        + ob_ref[...])
    oh16 = x[:, 39:55]

    w1 = w1_ref[...]
    ta = lax.dot_general(at_ref[...], w1[:, 4 * D:5 * D], dn,
                         preferred_element_type=f32)
    tg = lax.dot_general(gt_ref[...], w1[:, 5 * D:6 * D], dn,
                         preferred_element_type=f32)
    t16 = jnp.concatenate((ta, tg), axis=0)
    ag = lax.dot_general(oh16, t16, (((1,), (0,)), ((), ())),
                         preferred_element_type=f32)

    x4 = jnp.concatenate((u_ref[...], m_ref[...], g, o), axis=1)
    h1 = jax.nn.relu(
        lax.dot_general(x4, w1[:, 0:4 * D], dn, preferred_element_type=f32)
        + ag + b1_ref[...])
    h2 = jax.nn.relu(
        lax.dot_general(h1, w2_ref[...], dn, preferred_element_type=f32)
        + b2_ref[...])
    w3 = w3_ref[...]
    rows = [
        lax.dot_general(w3, h2[r * 128:(r + 1) * 128, :], dn,
                        preferred_element_type=f32)
        for r in range(_OR)
    ]
    out_ref[...] = jnp.concatenate(rows, axis=0) + b3_ref[0, 0]


def _dense_call(seg, features, u, m, gw, gb2d, ow, ob2d,
                at8, gt8, w1, b12d, w2, b22d, w3, b32d):
    grid = (_SEG // _BB,)
    off = seg * (_SEG // _BB)

    def seg_row(i):
        return (off + i, 0)

    def row(i):
        return (i, 0)

    def full(i):
        return (0, 0)

    in_specs = [
        pl.BlockSpec((_BB, 55), seg_row),
        pl.BlockSpec((_BB, D), row),
        pl.BlockSpec((_BB, D), row),
        pl.BlockSpec((D, 18), full),
        pl.BlockSpec((1, D), full),
        pl.BlockSpec((D, 21), full),
        pl.BlockSpec((1, D), full),
        pl.BlockSpec((8, D), full),
        pl.BlockSpec((8, D), full),
        pl.BlockSpec((256, 6 * D), full),
        pl.BlockSpec((1, 256), full),
        pl.BlockSpec((D, 256), full),
        pl.BlockSpec((1, D), full),
        pl.BlockSpec((1, D), full),
        pl.BlockSpec(memory_space=pltpu.SMEM),
    ]
    out_spec = pl.BlockSpec((_OR, 128), row)
    return pl.pallas_call(
        _dense_body,
        grid=grid,
        in_specs=in_specs,
        out_specs=out_spec,
        out_shape=jax.ShapeDtypeStruct((_SEG // 128, 128), jnp.float32),
    )(features, u, m, gw, gb2d, ow, ob2d,
      at8, gt8, w1, b12d, w2, b22d, w3, b32d)


def kernel(user_ids, movie_ids, genres, occupations, ages, genders,
           user_table, movie_table, age_table, gender_table,
           genre_W, genre_b, occ_W, occ_b,
           fc1_W, fc1_b, fc2_W, fc2_b, fc3_W, fc3_b):
    uids2d = user_ids.astype(jnp.int32).reshape(B // _CHUNK, _CHUNK)
    mids2d = movie_ids.astype(jnp.int32).reshape(B // _CHUNK, _CHUNK)

    features = jnp.concatenate(
        (genres.astype(jnp.bfloat16), occupations.astype(jnp.bfloat16),
         jax.nn.one_hot(ages, 8, dtype=jnp.bfloat16),
         jax.nn.one_hot(genders, 8, dtype=jnp.bfloat16)), axis=1)
    at8 = jnp.pad(age_table, ((0, 1), (0, 0)))
    gt8 = jnp.pad(gender_table, ((0, 6), (0, 0)))

    gathered = [
        _sc_gather(s, uids2d, mids2d, user_table, movie_table)
        for s in range(_NSEG)
    ]
    outs = [
        _dense_call(
            s, features, gathered[s][0], gathered[s][1],
            genre_W, genre_b.reshape(1, D), occ_W, occ_b.reshape(1, D),
            at8, gt8,
            fc1_W, fc1_b.reshape(1, 256), fc2_W, fc2_b.reshape(1, D),
            fc3_W, fc3_b.reshape(1, 1),
        )
        for s in range(_NSEG)
    ]
    return jnp.concatenate(outs, axis=0).reshape(B)


# speedup vs baseline: 1.0126x; 1.0126x over previous
import functools

import jax
import jax.numpy as jnp
from jax import lax
from jax.experimental import pallas as pl
from jax.experimental.pallas import tpu as pltpu
from jax.experimental.pallas import tpu_sc as plsc

B = 16384
D = 128


_NC = 2
_NS = 16
_NW = _NC * _NS
_CHUNK = 128
_NSEG = 2
_SEG = B // _NSEG
_CHUNKS_PER_W = _SEG // _NW // _CHUNK


def _sc_gather_body(seg, uids_hbm, mids_hbm, ut_hbm, mt_hbm,
                    uout_hbm, mout_hbm,
                    idxu, idxm, buf0, buf1, buf2, buf3, gsem, wsem):
    wid = lax.axis_index("s") * _NC + lax.axis_index("c")
    row0 = seg * (_SEG // _CHUNK) + wid * _CHUNKS_PER_W

    pltpu.sync_copy(uids_hbm.at[pl.ds(row0, _CHUNKS_PER_W)], idxu)
    pltpu.sync_copy(mids_hbm.at[pl.ds(row0, _CHUNKS_PER_W)], idxm)

    bufs = (buf0, buf1, buf2, buf3)
    rounds = []
    for tbl, idx, out in ((ut_hbm, idxu, uout_hbm), (mt_hbm, idxm, mout_hbm)):
        for g in range(_CHUNKS_PER_W):
            rounds.append((tbl, idx, g, out))

    gathers = [
        pltpu.async_copy(tbl.at[idx.at[g]], bufs[r], gsem)
        for r, (tbl, idx, g, _) in enumerate(rounds)
    ]
    writes = []
    for r, (_, _, g, out) in enumerate(rounds):
        gathers[r].wait()
        orow = (wid * _CHUNKS_PER_W + g) * _CHUNK
        writes.append(
            pltpu.async_copy(bufs[r], out.at[pl.ds(orow, _CHUNK)], wsem))
    for w in writes:
        w.wait()


def _sc_gather(seg, uids2d, mids2d, user_table, movie_table):
    mesh = plsc.VectorSubcoreMesh(core_axis_name="c", subcore_axis_name="s")
    k = functools.partial(
        pl.kernel,
        mesh=mesh,
        out_type=[
            jax.ShapeDtypeStruct((_SEG, D), jnp.float32),
            jax.ShapeDtypeStruct((_SEG, D), jnp.float32),
        ],
        scratch_types=[
            pltpu.VMEM((_CHUNKS_PER_W, _CHUNK), jnp.int32),
            pltpu.VMEM((_CHUNKS_PER_W, _CHUNK), jnp.int32),
            pltpu.VMEM((_CHUNK, D), jnp.float32),
            pltpu.VMEM((_CHUNK, D), jnp.float32),
            pltpu.VMEM((_CHUNK, D), jnp.float32),
            pltpu.VMEM((_CHUNK, D), jnp.float32),
            pltpu.SemaphoreType.DMA,
            pltpu.SemaphoreType.DMA,
        ],
    )(functools.partial(_sc_gather_body, seg))
    return k(uids2d, mids2d, user_table, movie_table)


_BB = 4096
_OR = _BB // 128


def _dense_body(f_ref, u_ref, m_ref,
                gw_ref, gb_ref, ow_ref, ob_ref, at_ref, gt_ref,
                w1_ref, b1_ref, w2_ref, b2_ref, w3_ref, b3_ref, out_ref):
    f32 = jnp.float32
    dn = (((1,), (1,)), ((), ()))

    x = f_ref[...].astype(f32)
    g = jax.nn.relu(
        lax.dot_general(x[:, 0:18], gw_ref[...], dn, preferred_element_type=f32)
        + gb_ref[...])
    o = jax.nn.relu(
        lax.dot_general(x[:, 18:39], ow_ref[...], dn, preferred_element_type=f32)
        + ob_ref[...])
    oh16 = x[:, 39:55]

    w1 = w1_ref[...]
    ta = lax.dot_general(at_ref[...], w1[:, 4 * D:5 * D], dn,
                         preferred_element_type=f32)
    tg = lax.dot_general(gt_ref[...], w1[:, 5 * D:6 * D], dn,
                         preferred_element_type=f32)
    t16 = jnp.concatenate((ta, tg), axis=0)
    ag = lax.dot_general(oh16, t16, (((1,), (0,)), ((), ())),
                         preferred_element_type=f32)

    x4 = jnp.concatenate((u_ref[...], m_ref[...], g, o), axis=1)
    h1 = jax.nn.relu(
        lax.dot_general(x4, w1[:, 0:4 * D], dn, preferred_element_type=f32)
        + ag + b1_ref[...])
    h2 = jax.nn.relu(
        lax.dot_general(h1, w2_ref[...], dn, preferred_element_type=f32)
        + b2_ref[...])
    w3 = w3_ref[...]
    rows = [
        lax.dot_general(w3, h2[r * 128:(r + 1) * 128, :], dn,
                        preferred_element_type=f32)
        for r in range(_OR)
    ]
    out_ref[...] = jnp.concatenate(rows, axis=0) + b3_ref[0, 0]


def _dense_call(seg, features, u, m, gw, gb2d, ow, ob2d,
                at8, gt8, w1, b12d, w2, b22d, w3, b32d):
    grid = (_SEG // _BB,)
    off = seg * (_SEG // _BB)

    def seg_row(i):
        return (off + i, 0)

    def row(i):
        return (i, 0)

    def full(i):
        return (0, 0)

    in_specs = [
        pl.BlockSpec((_BB, 55), seg_row),
        pl.BlockSpec((_BB, D), row),
        pl.BlockSpec((_BB, D), row),
        pl.BlockSpec((D, 18), full),
        pl.BlockSpec((1, D), full),
        pl.BlockSpec((D, 21), full),
        pl.BlockSpec((1, D), full),
        pl.BlockSpec((8, D), full),
        pl.BlockSpec((8, D), full),
        pl.BlockSpec((256, 6 * D), full),
        pl.BlockSpec((1, 256), full),
        pl.BlockSpec((D, 256), full),
        pl.BlockSpec((1, D), full),
        pl.BlockSpec((1, D), full),
        pl.BlockSpec(memory_space=pltpu.SMEM),
    ]
    out_spec = pl.BlockSpec((_OR, 128), row)
    return pl.pallas_call(
        _dense_body,
        grid=grid,
        in_specs=in_specs,
        out_specs=out_spec,
        out_shape=jax.ShapeDtypeStruct((_SEG // 128, 128), jnp.float32),
    )(features, u, m, gw, gb2d, ow, ob2d,
      at8, gt8, w1, b12d, w2, b22d, w3, b32d)


def kernel(user_ids, movie_ids, genres, occupations, ages, genders,
           user_table, movie_table, age_table, gender_table,
           genre_W, genre_b, occ_W, occ_b,
           fc1_W, fc1_b, fc2_W, fc2_b, fc3_W, fc3_b):
    uids2d = user_ids.astype(jnp.int32).reshape(B // _CHUNK, _CHUNK)
    mids2d = movie_ids.astype(jnp.int32).reshape(B // _CHUNK, _CHUNK)

    features = jnp.concatenate(
        (genres.astype(jnp.bfloat16), occupations.astype(jnp.bfloat16),
         jax.nn.one_hot(ages, 8, dtype=jnp.bfloat16),
         jax.nn.one_hot(genders, 8, dtype=jnp.bfloat16)), axis=1)
    at8 = jnp.pad(age_table, ((0, 1), (0, 0)))
    gt8 = jnp.pad(gender_table, ((0, 6), (0, 0)))

    gathered = [
        _sc_gather(s, uids2d, mids2d, user_table, movie_table)
        for s in range(_NSEG)
    ]
    outs = [
        _dense_call(
            s, features, gathered[s][0], gathered[s][1],
            genre_W, genre_b.reshape(1, D), occ_W, occ_b.reshape(1, D),
            at8, gt8,
            fc1_W, fc1_b.reshape(1, 256), fc2_W, fc2_b.reshape(1, D),
            fc3_W, fc3_b.reshape(1, 1),
        )
        for s in range(_NSEG)
    ]
    return jnp.concatenate(outs, axis=0).reshape(B)


# speedup vs baseline: 1.0842x; 1.0707x over previous
import functools

import jax
import jax.numpy as jnp
from jax import lax
from jax.experimental import pallas as pl
from jax.experimental.pallas import tpu as pltpu
from jax.experimental.pallas import tpu_sc as plsc

B = 16384
D = 128


_NC = 2
_NS = 16
_NW = _NC * _NS
_CHUNK = 128
_NSEG = 2
_SEG = B // _NSEG
_CHUNKS_PER_W = _SEG // _NW // _CHUNK


def _sc_gather_body(seg, uids_hbm, mids_hbm, ut_hbm, mt_hbm,
                    uout_hbm, mout_hbm,
                    idxu, idxm, buf0, buf1, buf2, buf3, gsem, wsem):
    wid = lax.axis_index("s") * _NC + lax.axis_index("c")
    row0 = seg * (_SEG // _CHUNK) + wid * _CHUNKS_PER_W

    pltpu.sync_copy(uids_hbm.at[pl.ds(row0, _CHUNKS_PER_W)], idxu)
    pltpu.sync_copy(mids_hbm.at[pl.ds(row0, _CHUNKS_PER_W)], idxm)

    bufs = (buf0, buf1, buf2, buf3)
    rounds = []
    for tbl, idx, out in ((ut_hbm, idxu, uout_hbm), (mt_hbm, idxm, mout_hbm)):
        for g in range(_CHUNKS_PER_W):
            rounds.append((tbl, idx, g, out))

    gathers = [
        pltpu.async_copy(tbl.at[idx.at[g]], bufs[r], gsem)
        for r, (tbl, idx, g, _) in enumerate(rounds)
    ]
    writes = []
    for r, (_, _, g, out) in enumerate(rounds):
        gathers[r].wait()
        orow = (wid * _CHUNKS_PER_W + g) * _CHUNK
        writes.append(
            pltpu.async_copy(bufs[r], out.at[pl.ds(orow, _CHUNK)], wsem))
    for w in writes:
        w.wait()


def _sc_gather(seg, uids2d, mids2d, user_table, movie_table):
    mesh = plsc.VectorSubcoreMesh(core_axis_name="c", subcore_axis_name="s")
    k = functools.partial(
        pl.kernel,
        mesh=mesh,
        out_type=[
            jax.ShapeDtypeStruct((_SEG, D), jnp.float32),
            jax.ShapeDtypeStruct((_SEG, D), jnp.float32),
        ],
        scratch_types=[
            pltpu.VMEM((_CHUNKS_PER_W, _CHUNK), jnp.int32),
            pltpu.VMEM((_CHUNKS_PER_W, _CHUNK), jnp.int32),
            pltpu.VMEM((_CHUNK, D), jnp.float32),
            pltpu.VMEM((_CHUNK, D), jnp.float32),
            pltpu.VMEM((_CHUNK, D), jnp.float32),
            pltpu.VMEM((_CHUNK, D), jnp.float32),
            pltpu.SemaphoreType.DMA,
            pltpu.SemaphoreType.DMA,
        ],
    )(functools.partial(_sc_gather_body, seg))
    return k(uids2d, mids2d, user_table, movie_table)


_BB = 4096
_OR = _BB // 128


def _dense_body(f_ref, u_ref, m_ref,
                gw_ref, gb_ref, ow_ref, ob_ref, at_ref, gt_ref,
                w1_ref, b1_ref, w2_ref, b2_ref, w3_ref, b3_ref, out_ref):
    f32 = jnp.float32
    dn = (((1,), (1,)), ((), ()))

    x = f_ref[...].astype(f32)
    g = jax.nn.relu(
        lax.dot_general(x[:, 0:18], gw_ref[...], dn, preferred_element_type=f32)
        + gb_ref[...])
    o = jax.nn.relu(
        lax.dot_general(x[:, 18:39], ow_ref[...], dn, preferred_element_type=f32)
        + ob_ref[...])
    oh16 = x[:, 39:55]

    w1 = w1_ref[...]
    ta = lax.dot_general(at_ref[...], w1[:, 4 * D:5 * D], dn,
                         preferred_element_type=f32)
    tg = lax.dot_general(gt_ref[...], w1[:, 5 * D:6 * D], dn,
                         preferred_element_type=f32)
    t16 = jnp.concatenate((ta, tg), axis=0)
    ag = lax.dot_general(oh16, t16, (((1,), (0,)), ((), ())),
                         preferred_element_type=f32)

    bf16 = jnp.bfloat16
    x4 = jnp.concatenate((u_ref[...], m_ref[...], g, o), axis=1).astype(bf16)
    h1 = jax.nn.relu(
        lax.dot_general(x4, w1[:, 0:4 * D].astype(bf16), dn,
                        preferred_element_type=f32)
        + ag + b1_ref[...])
    h2 = jax.nn.relu(
        lax.dot_general(h1.astype(bf16), w2_ref[...].astype(bf16), dn,
                        preferred_element_type=f32)
        + b2_ref[...])
    w3 = w3_ref[...]
    rows = [
        lax.dot_general(w3, h2[r * 128:(r + 1) * 128, :], dn,
                        preferred_element_type=f32)
        for r in range(_OR)
    ]
    out_ref[...] = jnp.concatenate(rows, axis=0) + b3_ref[0, 0]


def _dense_call(seg, features, u, m, gw, gb2d, ow, ob2d,
                at8, gt8, w1, b12d, w2, b22d, w3, b32d):
    grid = (_SEG // _BB,)
    off = seg * (_SEG // _BB)

    def seg_row(i):
        return (off + i, 0)

    def row(i):
        return (i, 0)

    def full(i):
        return (0, 0)

    in_specs = [
        pl.BlockSpec((_BB, 55), seg_row),
        pl.BlockSpec((_BB, D), row),
        pl.BlockSpec((_BB, D), row),
        pl.BlockSpec((D, 18), full),
        pl.BlockSpec((1, D), full),
        pl.BlockSpec((D, 21), full),
        pl.BlockSpec((1, D), full),
        pl.BlockSpec((8, D), full),
        pl.BlockSpec((8, D), full),
        pl.BlockSpec((256, 6 * D), full),
        pl.BlockSpec((1, 256), full),
        pl.BlockSpec((D, 256), full),
        pl.BlockSpec((1, D), full),
        pl.BlockSpec((1, D), full),
        pl.BlockSpec(memory_space=pltpu.SMEM),
    ]
    out_spec = pl.BlockSpec((_OR, 128), row)
    return pl.pallas_call(
        _dense_body,
        grid=grid,
        in_specs=in_specs,
        out_specs=out_spec,
        out_shape=jax.ShapeDtypeStruct((_SEG // 128, 128), jnp.float32),
    )(features, u, m, gw, gb2d, ow, ob2d,
      at8, gt8, w1, b12d, w2, b22d, w3, b32d)


def kernel(user_ids, movie_ids, genres, occupations, ages, genders,
           user_table, movie_table, age_table, gender_table,
           genre_W, genre_b, occ_W, occ_b,
           fc1_W, fc1_b, fc2_W, fc2_b, fc3_W, fc3_b):
    uids2d = user_ids.astype(jnp.int32).reshape(B // _CHUNK, _CHUNK)
    mids2d = movie_ids.astype(jnp.int32).reshape(B // _CHUNK, _CHUNK)

    features = jnp.concatenate(
        (genres.astype(jnp.bfloat16), occupations.astype(jnp.bfloat16),
         jax.nn.one_hot(ages, 8, dtype=jnp.bfloat16),
         jax.nn.one_hot(genders, 8, dtype=jnp.bfloat16)), axis=1)
    at8 = jnp.pad(age_table, ((0, 1), (0, 0)))
    gt8 = jnp.pad(gender_table, ((0, 6), (0, 0)))

    gathered = [
        _sc_gather(s, uids2d, mids2d, user_table, movie_table)
        for s in range(_NSEG)
    ]
    outs = [
        _dense_call(
            s, features, gathered[s][0], gathered[s][1],
            genre_W, genre_b.reshape(1, D), occ_W, occ_b.reshape(1, D),
            at8, gt8,
            fc1_W, fc1_b.reshape(1, 256), fc2_W, fc2_b.reshape(1, D),
            fc3_W, fc3_b.reshape(1, 1),
        )
        for s in range(_NSEG)
    ]
    return jnp.concatenate(outs, axis=0).reshape(B)
